# baseline (speedup 1.0000x reference)
import functools

import jax
import jax.numpy as jnp
from jax.experimental import pallas as pl
from jax.experimental.pallas import tpu as pltpu

_BM = 512
_FP = 16


def _body(nbatch, n, nm, xt_ref, adj_ref, adj4_ref, w1t_ref, b1_ref,
          s1_ref, t1_ref, wst_ref, bst_ref, sst_ref, tst_ref, out_ref,
          adjt_ref, h1_ref, hc_ref, y1_ref, buf_ref, sems):
    bp = pl.program_id(0)
    m = pl.program_id(1)
    parity = jax.lax.rem(bp, 2)
    chain_parity = jax.lax.rem(bp + 1, 2)
    mpb = nm - 1
    c = bp * mpb + jnp.minimum(m, mpb - 1)

    def issue(c_):
        b2 = c_ // mpb
        m2 = jax.lax.rem(c_, mpb)
        sl = jax.lax.rem(c_, 3)
        pltpu.make_async_copy(
            adj_ref.at[b2, pl.ds(m2 * _BM, _BM), :],
            buf_ref.at[sl], sems.at[sl]).start()

    @pl.when((bp == 0) & (m == 0))
    def _():
        issue(c)
        issue(c + 1)

    @pl.when(m < mpb)
    def _():
        sl = jax.lax.rem(c, 3)
        pltpu.make_async_copy(
            adj_ref.at[bp, pl.ds(jnp.minimum(m, mpb - 1) * _BM, _BM), :],
            buf_ref.at[sl], sems.at[sl]).wait()

        @pl.when(c + 2 < nbatch * mpb)
        def _():
            issue(c + 2)

    @pl.when((bp >= 1) & (m == 0))
    def _():
        hc_ref[...] = h1_ref[...]

    @pl.when(m == 0)
    def _():
        y1n = jnp.dot(xt_ref[0], w1t_ref[...],
                      preferred_element_type=jnp.float32)
        y1_ref[...] = jnp.swapaxes(y1n, 0, 1).astype(jnp.bfloat16)

    def stream(src):
        at = jnp.swapaxes(src, 0, 1).astype(jnp.bfloat16)
        adjt_ref[parity, :, pl.ds(m * _BM, _BM)] = at
        acc1 = jnp.dot(y1_ref[...], at, preferred_element_type=jnp.float32)
        h1_ref[:, pl.ds(m * _BM, _BM)] = (
            (acc1 + b1_ref[...]) * s1_ref[...] + t1_ref[...])

    @pl.when(m < mpb)
    def _():
        stream(buf_ref[jax.lax.rem(c, 3)])

    @pl.when(m == mpb)
    def _():
        stream(adj4_ref[0])

    def chain_layer(layer, cpar):
        y = jnp.dot(wst_ref[layer], hc_ref[...],
                    preferred_element_type=jnp.float32
                    )[:, 0:n].astype(jnp.bfloat16)
        acc = jnp.dot(y, adjt_ref[cpar], preferred_element_type=jnp.float32)
        hc_ref[...] = (acc + bst_ref[layer]) * sst_ref[layer] + tst_ref[layer]

    def emit(bc):
        out_ref[bc, :, :] = hc_ref[0:7, 0:n]

    lps = -(-9 // nm)
    for j in range(lps):

        @pl.when((bp >= 1) & (m * lps + j < 9))
        def _(j=j):
            chain_layer(m * lps + j, chain_parity)

    @pl.when((bp >= 1) & (m == nm - 1))
    def _():
        emit(bp - 1)

    @pl.when((bp == nbatch - 1) & (m == nm - 1))
    def _():
        hc_ref[...] = h1_ref[...]

        def drain(l, c):
            chain_layer(l, parity)
            return c

        jax.lax.fori_loop(0, 9, drain, 0)
        emit(bp)


def kernel(x, adj, W1, b1, W2, b2, W3, b3, W4, b4, W5, b5, W6, b6, W7, b7,
           W8, b8, W9, b9, W10, b10, g1, beta1, g2, beta2, g3, beta3,
           g4, beta4, g5, beta5, g6, beta6, g7, beta7):
    bsz, n, f0 = x.shape
    nm = pl.cdiv(n, _BM)
    wpad = nm * _BM
    ws = [W1, W2, W3, W4, W5, W6, W7, W8, W9, W10]
    bs = [b1, b2, b3, b4, b5, b6, b7, b8, b9, b10]
    gs = [g1, g2, g3, g4, g5, g6, g7]
    bes = [beta1, beta2, beta3, beta4, beta5, beta6, beta7]
    inv = 1.0 / jnp.sqrt(jnp.float32(1.0 + 1e-5))
    ones = jnp.ones((n,), jnp.float32)
    zeros = jnp.zeros((n,), jnp.float32)

    w1t = jnp.pad(W1, ((0, 0), (0, _FP - W1.shape[1])))
    b1c = jnp.pad(b1, (0, _FP - b1.shape[0])).reshape(_FP, 1)
    s1 = (gs[0] * inv).reshape(1, n)
    t1 = bes[0].reshape(1, n)

    wst = jnp.stack([
        jnp.pad(ws[i].T, ((0, _FP - ws[i].shape[1]), (0, _FP - ws[i].shape[0])))
        for i in range(1, 10)])
    bst = jnp.stack([jnp.pad(bs[i], (0, _FP - bs[i].shape[0]))
                     for i in range(1, 10)]).reshape(9, _FP, 1)
    sst = jnp.pad(
        jnp.stack([gs[i] * inv if i < 7 else ones for i in range(1, 10)]),
        ((0, 0), (0, wpad - n))).reshape(9, 1, wpad)
    tst = jnp.pad(
        jnp.stack([bes[i] if i < 7 else zeros for i in range(1, 10)]),
        ((0, 0), (0, wpad - n))).reshape(9, 1, wpad)

    body = functools.partial(_body, bsz, n, nm)
    outt = pl.pallas_call(
        body,
        grid=(bsz, nm),
        in_specs=[
            pl.BlockSpec((1, n, f0), lambda b, m: (b, 0, 0)),
            pl.BlockSpec(memory_space=pl.ANY),
            pl.BlockSpec((1, _BM, n), lambda b, m, q=nm - 1: (b, q, 0)),
            pl.BlockSpec((f0, _FP), lambda b, m: (0, 0)),
            pl.BlockSpec((_FP, 1), lambda b, m: (0, 0)),
            pl.BlockSpec((1, _BM), lambda b, m: (0, m)),
            pl.BlockSpec((1, _BM), lambda b, m: (0, m)),
            pl.BlockSpec((9, _FP, _FP), lambda b, m: (0, 0, 0)),
            pl.BlockSpec((9, _FP, 1), lambda b, m: (0, 0, 0)),
            pl.BlockSpec((9, 1, wpad), lambda b, m: (0, 0, 0)),
            pl.BlockSpec((9, 1, wpad), lambda b, m: (0, 0, 0)),
        ],
        out_specs=pl.BlockSpec((bsz, 7, n), lambda b, m: (0, 0, 0)),
        out_shape=jax.ShapeDtypeStruct((bsz, 7, n), jnp.float32),
        scratch_shapes=[
            pltpu.VMEM((2, n, wpad), jnp.bfloat16),
            pltpu.VMEM((_FP, wpad), jnp.float32),
            pltpu.VMEM((_FP, wpad), jnp.float32),
            pltpu.VMEM((_FP, n), jnp.bfloat16),
            pltpu.VMEM((3, _BM, n), jnp.float32),
            pltpu.SemaphoreType.DMA((3,)),
        ],
    )(x, adj, adj, w1t, b1c, s1, t1, wst, bst, sst, tst)
    return jnp.swapaxes(outt, 1, 2)


# speedup vs baseline: 1.0025x; 1.0025x over previous
import functools

import jax
import jax.numpy as jnp
from jax.experimental import pallas as pl
from jax.experimental.pallas import tpu as pltpu

_BM = 512
_FP = 16


def _body(nbatch, n, nm, xt_ref, adj_ref, adj4_ref, w1t_ref, b1_ref,
          s1_ref, t1_ref, wst_ref, bst_ref, sst_ref, tst_ref, out_ref,
          adjt_ref, h1_ref, hc_ref, y1_ref, buf_ref, sems):
    bp = pl.program_id(0)
    m = pl.program_id(1)
    parity = jax.lax.rem(bp, 2)
    chain_parity = jax.lax.rem(bp + 1, 2)
    mpb = nm - 1
    c = bp * mpb + jnp.minimum(m, mpb - 1)

    def issue(c_):
        b2 = c_ // mpb
        m2 = jax.lax.rem(c_, mpb)
        sl = jax.lax.rem(c_, 3)
        pltpu.make_async_copy(
            adj_ref.at[b2, pl.ds(m2 * _BM, _BM), :],
            buf_ref.at[sl], sems.at[sl]).start()

    @pl.when((bp == 0) & (m == 0))
    def _():
        issue(c)
        issue(c + 1)

    @pl.when(m < mpb)
    def _():
        sl = jax.lax.rem(c, 3)
        pltpu.make_async_copy(
            adj_ref.at[bp, pl.ds(jnp.minimum(m, mpb - 1) * _BM, _BM), :],
            buf_ref.at[sl], sems.at[sl]).wait()

        @pl.when(c + 2 < nbatch * mpb)
        def _():
            issue(c + 2)

    @pl.when((bp >= 1) & (m == 0))
    def _():
        hc_ref[...] = h1_ref[...]

    @pl.when(m == 0)
    def _():
        y1n = jnp.dot(xt_ref[0], w1t_ref[...],
                      preferred_element_type=jnp.float32)
        y1_ref[...] = jnp.swapaxes(y1n, 0, 1).astype(jnp.bfloat16)

    def stream(src):
        at = jnp.swapaxes(src, 0, 1).astype(jnp.bfloat16)
        adjt_ref[parity, :, pl.ds(m * _BM, _BM)] = at
        acc1 = jnp.dot(y1_ref[...], at, preferred_element_type=jnp.float32)
        h1_ref[:, pl.ds(m * _BM, _BM)] = (
            (acc1 + b1_ref[...]) * s1_ref[...] + t1_ref[...])

    @pl.when(m < mpb)
    def _():
        stream(buf_ref[jax.lax.rem(c, 3)])

    @pl.when(m == mpb)
    def _():
        stream(buf_ref[jax.lax.rem(c, 3)])

    def chain_layer(layer, cpar):
        y = jnp.dot(wst_ref[layer], hc_ref[...],
                    preferred_element_type=jnp.float32
                    )[:, 0:n].astype(jnp.bfloat16)
        acc = jnp.dot(y, adjt_ref[cpar], preferred_element_type=jnp.float32)
        hc_ref[...] = (acc + bst_ref[layer]) * sst_ref[layer] + tst_ref[layer]

    def emit(bc):
        out_ref[bc, :, :] = hc_ref[0:7, 0:n]

    lps = -(-9 // nm)
    for j in range(lps):

        @pl.when((bp >= 1) & (m * lps + j < 9))
        def _(j=j):
            chain_layer(m * lps + j, chain_parity)

    @pl.when((bp >= 1) & (m == nm - 1))
    def _():
        emit(bp - 1)

    @pl.when((bp == nbatch - 1) & (m == nm - 1))
    def _():
        hc_ref[...] = h1_ref[...]

        def drain(l, c):
            chain_layer(l, parity)
            return c

        jax.lax.fori_loop(0, 9, drain, 0)
        emit(bp)


def kernel(x, adj, W1, b1, W2, b2, W3, b3, W4, b4, W5, b5, W6, b6, W7, b7,
           W8, b8, W9, b9, W10, b10, g1, beta1, g2, beta2, g3, beta3,
           g4, beta4, g5, beta5, g6, beta6, g7, beta7):
    bsz, n, f0 = x.shape
    nm = pl.cdiv(n, _BM)
    wpad = nm * _BM
    ws = [W1, W2, W3, W4, W5, W6, W7, W8, W9, W10]
    bs = [b1, b2, b3, b4, b5, b6, b7, b8, b9, b10]
    gs = [g1, g2, g3, g4, g5, g6, g7]
    bes = [beta1, beta2, beta3, beta4, beta5, beta6, beta7]
    inv = 1.0 / jnp.sqrt(jnp.float32(1.0 + 1e-5))
    ones = jnp.ones((n,), jnp.float32)
    zeros = jnp.zeros((n,), jnp.float32)

    w1t = jnp.pad(W1, ((0, 0), (0, _FP - W1.shape[1])))
    b1c = jnp.pad(b1, (0, _FP - b1.shape[0])).reshape(_FP, 1)
    s1 = (gs[0] * inv).reshape(1, n)
    t1 = bes[0].reshape(1, n)

    wst = jnp.stack([
        jnp.pad(ws[i].T, ((0, _FP - ws[i].shape[1]), (0, _FP - ws[i].shape[0])))
        for i in range(1, 10)])
    bst = jnp.stack([jnp.pad(bs[i], (0, _FP - bs[i].shape[0]))
                     for i in range(1, 10)]).reshape(9, _FP, 1)
    sst = jnp.pad(
        jnp.stack([gs[i] * inv if i < 7 else ones for i in range(1, 10)]),
        ((0, 0), (0, wpad - n))).reshape(9, 1, wpad)
    tst = jnp.pad(
        jnp.stack([bes[i] if i < 7 else zeros for i in range(1, 10)]),
        ((0, 0), (0, wpad - n))).reshape(9, 1, wpad)

    body = functools.partial(_body, bsz, n, nm)
    outt = pl.pallas_call(
        body,
        grid=(bsz, nm),
        in_specs=[
            pl.BlockSpec((1, n, f0), lambda b, m: (b, 0, 0)),
            pl.BlockSpec(memory_space=pl.ANY),
            pl.BlockSpec((1, 16, 128), lambda b, m: (0, 0, 0)),
            pl.BlockSpec((f0, _FP), lambda b, m: (0, 0)),
            pl.BlockSpec((_FP, 1), lambda b, m: (0, 0)),
            pl.BlockSpec((1, _BM), lambda b, m: (0, m)),
            pl.BlockSpec((1, _BM), lambda b, m: (0, m)),
            pl.BlockSpec((9, _FP, _FP), lambda b, m: (0, 0, 0)),
            pl.BlockSpec((9, _FP, 1), lambda b, m: (0, 0, 0)),
            pl.BlockSpec((9, 1, wpad), lambda b, m: (0, 0, 0)),
            pl.BlockSpec((9, 1, wpad), lambda b, m: (0, 0, 0)),
        ],
        out_specs=pl.BlockSpec((bsz, 7, n), lambda b, m: (0, 0, 0)),
        out_shape=jax.ShapeDtypeStruct((bsz, 7, n), jnp.float32),
        scratch_shapes=[
            pltpu.VMEM((2, n, wpad), jnp.bfloat16),
            pltpu.VMEM((_FP, wpad), jnp.float32),
            pltpu.VMEM((_FP, wpad), jnp.float32),
            pltpu.VMEM((_FP, n), jnp.bfloat16),
            pltpu.VMEM((3, _BM, n), jnp.float32),
            pltpu.SemaphoreType.DMA((3,)),
        ],
    )(x, adj, adj[0:1, 0:16, 0:128], w1t, b1c, s1, t1, wst, bst, sst, tst)
    return jnp.swapaxes(outt, 1, 2)


# speedup vs baseline: 1.1532x; 1.1504x over previous
import functools

import jax
import jax.numpy as jnp
from jax.experimental import pallas as pl
from jax.experimental.pallas import tpu as pltpu

_BM = 512
_FP = 16


def _body(nbatch, n, nm, xt_ref, adj_ref, adj4_ref, w1t_ref, b1_ref,
          s1_ref, t1_ref, wst_ref, bst_ref, sst_ref, tst_ref, out_ref,
          adjt_ref, h1_ref, hc_ref, y1_ref, buf_ref, sems):
    bp = pl.program_id(0)
    m = pl.program_id(1)
    parity = jax.lax.rem(bp, 2)
    chain_parity = jax.lax.rem(bp + 1, 2)
    mpb = nm - 1
    c = bp * mpb + jnp.minimum(m, mpb - 1)

    def issue(c_):
        b2 = c_ // mpb
        m2 = jax.lax.rem(c_, mpb)
        sl = jax.lax.rem(c_, 3)
        pltpu.make_async_copy(
            adj_ref.at[b2, pl.ds(m2 * _BM, _BM), :],
            buf_ref.at[sl], sems.at[sl]).start()

    @pl.when((bp == 0) & (m == 0))
    def _():
        issue(c)
        issue(c + 1)

    @pl.when(m < mpb)
    def _():
        sl = jax.lax.rem(c, 3)
        pltpu.make_async_copy(
            adj_ref.at[bp, pl.ds(jnp.minimum(m, mpb - 1) * _BM, _BM), :],
            buf_ref.at[sl], sems.at[sl]).wait()

        @pl.when(c + 2 < nbatch * mpb)
        def _():
            issue(c + 2)

    @pl.when((bp >= 1) & (m == 0))
    def _():
        hc_ref[...] = h1_ref[...]

    @pl.when(m == 0)
    def _():
        y1n = jnp.dot(xt_ref[0], w1t_ref[...],
                      preferred_element_type=jnp.float32)
        y1_ref[...] = jnp.swapaxes(y1n, 0, 1).astype(jnp.bfloat16)

    def stream(src):
        at = jnp.swapaxes(src, 0, 1)
        adjt_ref[parity, :, pl.ds(m * _BM, _BM)] = at
        acc1 = jnp.dot(y1_ref[...], at, preferred_element_type=jnp.float32)
        h1_ref[:, pl.ds(m * _BM, _BM)] = (
            (acc1 + b1_ref[...]) * s1_ref[...] + t1_ref[...])

    @pl.when(m < mpb)
    def _():
        stream(buf_ref[jax.lax.rem(c, 3)])

    @pl.when(m == mpb)
    def _():
        stream(adj4_ref[0])

    def chain_layer(layer, cpar):
        y = jnp.dot(wst_ref[layer], hc_ref[...],
                    preferred_element_type=jnp.float32
                    )[:, 0:n].astype(jnp.bfloat16)
        acc = jnp.dot(y, adjt_ref[cpar], preferred_element_type=jnp.float32)
        hc_ref[...] = (acc + bst_ref[layer]) * sst_ref[layer] + tst_ref[layer]

    def emit(bc):
        out_ref[bc, :, :] = hc_ref[0:7, 0:n]

    lps = -(-9 // nm)
    for j in range(lps):

        @pl.when((bp >= 1) & (m * lps + j < 9))
        def _(j=j):
            chain_layer(m * lps + j, chain_parity)

    @pl.when((bp >= 1) & (m == nm - 1))
    def _():
        emit(bp - 1)

    @pl.when((bp == nbatch - 1) & (m == nm - 1))
    def _():
        hc_ref[...] = h1_ref[...]

        def drain(l, c):
            chain_layer(l, parity)
            return c

        jax.lax.fori_loop(0, 9, drain, 0)
        emit(bp)


def kernel(x, adj, W1, b1, W2, b2, W3, b3, W4, b4, W5, b5, W6, b6, W7, b7,
           W8, b8, W9, b9, W10, b10, g1, beta1, g2, beta2, g3, beta3,
           g4, beta4, g5, beta5, g6, beta6, g7, beta7):
    bsz, n, f0 = x.shape
    nm = pl.cdiv(n, _BM)
    wpad = nm * _BM
    ws = [W1, W2, W3, W4, W5, W6, W7, W8, W9, W10]
    bs = [b1, b2, b3, b4, b5, b6, b7, b8, b9, b10]
    gs = [g1, g2, g3, g4, g5, g6, g7]
    bes = [beta1, beta2, beta3, beta4, beta5, beta6, beta7]
    inv = 1.0 / jnp.sqrt(jnp.float32(1.0 + 1e-5))
    ones = jnp.ones((n,), jnp.float32)
    zeros = jnp.zeros((n,), jnp.float32)

    w1t = jnp.pad(W1, ((0, 0), (0, _FP - W1.shape[1])))
    b1c = jnp.pad(b1, (0, _FP - b1.shape[0])).reshape(_FP, 1)
    s1 = (gs[0] * inv).reshape(1, n)
    t1 = bes[0].reshape(1, n)

    wst = jnp.stack([
        jnp.pad(ws[i].T, ((0, _FP - ws[i].shape[1]), (0, _FP - ws[i].shape[0])))
        for i in range(1, 10)])
    bst = jnp.stack([jnp.pad(bs[i], (0, _FP - bs[i].shape[0]))
                     for i in range(1, 10)]).reshape(9, _FP, 1)
    sst = jnp.pad(
        jnp.stack([gs[i] * inv if i < 7 else ones for i in range(1, 10)]),
        ((0, 0), (0, wpad - n))).reshape(9, 1, wpad)
    tst = jnp.pad(
        jnp.stack([bes[i] if i < 7 else zeros for i in range(1, 10)]),
        ((0, 0), (0, wpad - n))).reshape(9, 1, wpad)

    adjb = adj.astype(jnp.bfloat16)
    tail0 = (nm - 1) * _BM
    adj4 = jnp.pad(adj[:, tail0:, :], ((0, 0), (0, wpad - n), (0, 0))
                   ).astype(jnp.bfloat16)

    body = functools.partial(_body, bsz, n, nm)
    outt = pl.pallas_call(
        body,
        grid=(bsz, nm),
        in_specs=[
            pl.BlockSpec((1, n, f0), lambda b, m: (b, 0, 0)),
            pl.BlockSpec(memory_space=pl.ANY),
            pl.BlockSpec((1, _BM, n), lambda b, m: (b, 0, 0)),
            pl.BlockSpec((f0, _FP), lambda b, m: (0, 0)),
            pl.BlockSpec((_FP, 1), lambda b, m: (0, 0)),
            pl.BlockSpec((1, _BM), lambda b, m: (0, m)),
            pl.BlockSpec((1, _BM), lambda b, m: (0, m)),
            pl.BlockSpec((9, _FP, _FP), lambda b, m: (0, 0, 0)),
            pl.BlockSpec((9, _FP, 1), lambda b, m: (0, 0, 0)),
            pl.BlockSpec((9, 1, wpad), lambda b, m: (0, 0, 0)),
            pl.BlockSpec((9, 1, wpad), lambda b, m: (0, 0, 0)),
        ],
        out_specs=pl.BlockSpec((bsz, 7, n), lambda b, m: (0, 0, 0)),
        out_shape=jax.ShapeDtypeStruct((bsz, 7, n), jnp.float32),
        scratch_shapes=[
            pltpu.VMEM((2, n, wpad), jnp.bfloat16),
            pltpu.VMEM((_FP, wpad), jnp.float32),
            pltpu.VMEM((_FP, wpad), jnp.float32),
            pltpu.VMEM((_FP, n), jnp.bfloat16),
            pltpu.VMEM((3, _BM, n), jnp.bfloat16),
            pltpu.SemaphoreType.DMA((3,)),
        ],
    )(x, adjb, adj4, w1t, b1c, s1, t1, wst, bst, sst, tst)
    return jnp.swapaxes(outt, 1, 2)
